# baseline (device time: 28192 ns/iter reference)
import jax
import jax.numpy as jnp
from jax import lax
from jax.experimental import pallas as pl
from jax.experimental.pallas import tpu as pltpu

B, S, H, D = 2, 256, 8, 64
HALF = S // 2
QTR = HALF // 2
SCALE = D ** -0.5
QCLIP = 5.0
QSCALE = 127.0 / QCLIP
DEQ = 1.0 / QSCALE


def kernel(Q, K, V):
    def body(q_hbm, k_hbm, v_hbm, o_hbm,
             qt, kt, vt, ot, snd, rcv_y, rcv_x, l_ref,
             in_sems, out_sems, sems_ys, sems_yr, sems_xs, sems_xr):
        my_x = lax.axis_index("x")
        my_y = lax.axis_index("y")
        nbr_y = (my_x, 1 - my_y)
        nbr_x = (1 - my_x, my_y)

        in_dmas = []
        for t, (src, dst) in enumerate(((k_hbm, kt), (v_hbm, vt),
                                        (q_hbm, qt))):
            for h in range(H):
                dma = pltpu.make_async_copy(
                    src.at[:, :, h, :], dst.at[:, h], in_sems.at[t * H + h]
                )
                dma.start()
                in_dmas.append(dma)

        barrier_sem = pltpu.get_barrier_semaphore()
        for nbr in (nbr_y, nbr_x):
            pl.semaphore_signal(
                barrier_sem, inc=1, device_id=nbr,
                device_id_type=pl.DeviceIdType.MESH,
            )
        pl.semaphore_wait(barrier_sem, 2)

        for dma in in_dmas[: 2 * H]:
            dma.wait()

        for t, ref in ((0, kt), (1, vt)):
            half = ref[:, :, pl.ds(my_x * HALF, HALF), :]
            snd[t] = jnp.round(
                jnp.clip(half * QSCALE, -127.0, 127.0)
            ).astype(jnp.int8)

        def chunk_copy(src, dst, c, send_sem, recv_sem, dev):
            t, sub = divmod(c, 2)
            sl = (t, slice(None), slice(None), pl.ds(sub * QTR, QTR))
            return pltpu.make_async_remote_copy(
                src_ref=src.at[sl], dst_ref=dst.at[sl],
                send_sem=send_sem, recv_sem=recv_sem,
                device_id=dev, device_id_type=pl.DeviceIdType.MESH,
            )

        rdma_y = [
            chunk_copy(snd, rcv_y, c, sems_ys.at[c], sems_yr.at[c], nbr_y)
            for c in range(4)
        ]
        for r in rdma_y:
            r.start()

        for dma in in_dmas[2 * H:]:
            dma.wait()

        def block(b, h, k_blk, v_blk, s_scale):
            q = qt[b, h].astype(jnp.bfloat16)
            s = lax.dot_general(
                q, k_blk, (((1,), (1,)), ((), ())),
                preferred_element_type=jnp.float32,
            ) * s_scale
            p = jnp.exp(s)
            l = jnp.sum(p, axis=-1, keepdims=True)
            o = lax.dot_general(
                p.astype(jnp.bfloat16), v_blk, (((1,), (0,)), ((), ())),
                preferred_element_type=jnp.float32,
            )
            return o, l

        for b in range(B):
            for h in range(H):
                o, l = block(b, h, kt[b, h].astype(jnp.bfloat16),
                             vt[b, h].astype(jnp.bfloat16), SCALE)
                ot[b, h] = o
                l_ref[b, h] = l

        rdma_x = [
            chunk_copy(rcv_y, rcv_x, c, sems_xs.at[c], sems_xr.at[c], nbr_x)
            for c in range(4)
        ]
        for ry, rx in zip(rdma_y, rdma_x):
            ry.wait_recv()
            rx.start()

        for b in range(B):
            for h in range(H):
                o, l = block(b, h, rcv_y[0, b, h].astype(jnp.bfloat16),
                             rcv_y[1, b, h].astype(jnp.bfloat16),
                             SCALE * DEQ)
                ot[b, h] += o * DEQ
                l_ref[b, h] += l

        for rx in rdma_x:
            rx.wait_recv()

        out_dmas = []
        for h in range(H):
            for b in range(B):
                o, l = block(b, h, rcv_x[0, b, h].astype(jnp.bfloat16),
                             rcv_x[1, b, h].astype(jnp.bfloat16),
                             SCALE * DEQ)
                ot[b, h] = (ot[b, h] + o * DEQ) * (1.0 / (l_ref[b, h] + l))
            dma = pltpu.make_async_copy(
                ot.at[:, h], o_hbm.at[:, :, h, :], out_sems.at[h]
            )
            dma.start()
            out_dmas.append(dma)

        for dma in out_dmas:
            dma.wait()

        for r in rdma_y + rdma_x:
            r.wait_send()

    return pl.pallas_call(
        body,
        out_shape=jax.ShapeDtypeStruct((B, S, H, D), jnp.float32),
        in_specs=[
            pl.BlockSpec(memory_space=pl.ANY),
            pl.BlockSpec(memory_space=pl.ANY),
            pl.BlockSpec(memory_space=pl.ANY),
        ],
        out_specs=pl.BlockSpec(memory_space=pl.ANY),
        scratch_shapes=[
            pltpu.VMEM((B, H, S, D), jnp.float32),
            pltpu.VMEM((B, H, S, D), jnp.float32),
            pltpu.VMEM((B, H, S, D), jnp.float32),
            pltpu.VMEM((B, H, S, D), jnp.float32),
            pltpu.VMEM((2, B, H, HALF, D), jnp.int8),
            pltpu.VMEM((2, B, H, HALF, D), jnp.int8),
            pltpu.VMEM((2, B, H, HALF, D), jnp.int8),
            pltpu.VMEM((B, H, S, 1), jnp.float32),
            pltpu.SemaphoreType.DMA((3 * H,)),
            pltpu.SemaphoreType.DMA((H,)),
            pltpu.SemaphoreType.DMA((4,)),
            pltpu.SemaphoreType.DMA((4,)),
            pltpu.SemaphoreType.DMA((4,)),
            pltpu.SemaphoreType.DMA((4,)),
        ],
        compiler_params=pltpu.CompilerParams(collective_id=0),
    )(Q, K, V)


# device time: 20133 ns/iter; 1.4003x vs baseline; 1.4003x over previous
import jax
import jax.numpy as jnp
from jax import lax
from jax.experimental import pallas as pl
from jax.experimental.pallas import tpu as pltpu

B, S, H, D = 2, 256, 8, 64
HALF = S // 2
QTR = HALF // 2
SCALE = D ** -0.5
QCLIP = 5.0
QSCALE = 127.0 / QCLIP
DEQ = 1.0 / QSCALE


def kernel(Q, K, V):
    Qt = jnp.transpose(Q, (0, 2, 1, 3))
    Kt = jnp.transpose(K, (0, 2, 1, 3))
    Vt = jnp.transpose(V, (0, 2, 1, 3))

    def body(q_ref, k_ref, v_ref, o_ref, snd, rcv_y, rcv_x, l_ref,
             sems_ys, sems_yr, sems_xs, sems_xr):
        my_x = lax.axis_index("x")
        my_y = lax.axis_index("y")
        nbr_y = (my_x, 1 - my_y)
        nbr_x = (1 - my_x, my_y)

        barrier_sem = pltpu.get_barrier_semaphore()
        for nbr in (nbr_y, nbr_x):
            pl.semaphore_signal(
                barrier_sem, inc=1, device_id=nbr,
                device_id_type=pl.DeviceIdType.MESH,
            )

        for t, ref in ((0, k_ref), (1, v_ref)):
            half = ref[:, :, pl.ds(my_x * HALF, HALF), :]
            snd[t] = jnp.round(
                jnp.clip(half * QSCALE, -127.0, 127.0)
            ).astype(jnp.int8)

        pl.semaphore_wait(barrier_sem, 2)

        def chunk_copy(src, dst, c, send_sem, recv_sem, dev):
            t, sub = divmod(c, 2)
            sl = (t, slice(None), slice(None), pl.ds(sub * QTR, QTR))
            return pltpu.make_async_remote_copy(
                src_ref=src.at[sl], dst_ref=dst.at[sl],
                send_sem=send_sem, recv_sem=recv_sem,
                device_id=dev, device_id_type=pl.DeviceIdType.MESH,
            )

        rdma_y = [
            chunk_copy(snd, rcv_y, c, sems_ys.at[c], sems_yr.at[c], nbr_y)
            for c in range(4)
        ]
        for r in rdma_y:
            r.start()

        def block(b, h, k_blk, v_blk, s_scale):
            q = q_ref[b, h].astype(jnp.bfloat16)
            s = lax.dot_general(
                q, k_blk, (((1,), (1,)), ((), ())),
                preferred_element_type=jnp.float32,
            ) * s_scale
            p = jnp.exp(s)
            l = jnp.sum(p, axis=-1, keepdims=True)
            o = lax.dot_general(
                p.astype(jnp.bfloat16), v_blk, (((1,), (0,)), ((), ())),
                preferred_element_type=jnp.float32,
            )
            return o, l

        for b in range(B):
            for h in range(H):
                o, l = block(b, h, k_ref[b, h].astype(jnp.bfloat16),
                             v_ref[b, h].astype(jnp.bfloat16), SCALE)
                o_ref[b, h] = o
                l_ref[b, h] = l

        rdma_x = [
            chunk_copy(rcv_y, rcv_x, c, sems_xs.at[c], sems_xr.at[c], nbr_x)
            for c in range(4)
        ]
        for ry, rx in zip(rdma_y, rdma_x):
            ry.wait_recv()
            rx.start()

        for b in range(B):
            for h in range(H):
                o, l = block(b, h, rcv_y[0, b, h].astype(jnp.bfloat16),
                             rcv_y[1, b, h].astype(jnp.bfloat16),
                             SCALE * DEQ)
                o_ref[b, h] += o * DEQ
                l_ref[b, h] += l

        for rx in rdma_x:
            rx.wait_recv()

        for b in range(B):
            for h in range(H):
                o, l = block(b, h, rcv_x[0, b, h].astype(jnp.bfloat16),
                             rcv_x[1, b, h].astype(jnp.bfloat16),
                             SCALE * DEQ)
                o_ref[b, h] = (o_ref[b, h] + o * DEQ) \
                    * (1.0 / (l_ref[b, h] + l))

        for r in rdma_y + rdma_x:
            r.wait_send()

    out_t = pl.pallas_call(
        body,
        out_shape=jax.ShapeDtypeStruct((B, H, S, D), jnp.float32),
        in_specs=[
            pl.BlockSpec(memory_space=pltpu.VMEM),
            pl.BlockSpec(memory_space=pltpu.VMEM),
            pl.BlockSpec(memory_space=pltpu.VMEM),
        ],
        out_specs=pl.BlockSpec(memory_space=pltpu.VMEM),
        scratch_shapes=[
            pltpu.VMEM((2, B, H, HALF, D), jnp.int8),
            pltpu.VMEM((2, B, H, HALF, D), jnp.int8),
            pltpu.VMEM((2, B, H, HALF, D), jnp.int8),
            pltpu.VMEM((B, H, S, 1), jnp.float32),
            pltpu.SemaphoreType.DMA((4,)),
            pltpu.SemaphoreType.DMA((4,)),
            pltpu.SemaphoreType.DMA((4,)),
            pltpu.SemaphoreType.DMA((4,)),
        ],
        compiler_params=pltpu.CompilerParams(collective_id=0),
    )(Qt, Kt, Vt)

    return jnp.transpose(out_t, (0, 2, 1, 3))


# device time: 19492 ns/iter; 1.4463x vs baseline; 1.0329x over previous
import jax
import jax.numpy as jnp
from jax import lax
from jax.experimental import pallas as pl
from jax.experimental.pallas import tpu as pltpu

B, S, H, D = 2, 256, 8, 64
HALF = S // 2
QTR = HALF // 2
SCALE = D ** -0.5
QCLIP = 5.0
QSCALE = 127.0 / QCLIP
DEQ = 1.0 / QSCALE


def kernel(Q, K, V):
    Qt = jnp.transpose(Q, (0, 2, 1, 3)).astype(jnp.bfloat16)
    Kt = jnp.transpose(K, (0, 2, 1, 3)).astype(jnp.bfloat16)
    Vt = jnp.transpose(V, (0, 2, 1, 3)).astype(jnp.bfloat16)

    def body(q_ref, k_ref, v_ref, o_ref, snd, rcv_y, rcv_x, l_ref,
             sems_ys, sems_yr, sems_xs, sems_xr):
        my_x = lax.axis_index("x")
        my_y = lax.axis_index("y")
        nbr_y = (my_x, 1 - my_y)
        nbr_x = (1 - my_x, my_y)

        barrier_sem = pltpu.get_barrier_semaphore()
        for nbr in (nbr_y, nbr_x):
            pl.semaphore_signal(
                barrier_sem, inc=1, device_id=nbr,
                device_id_type=pl.DeviceIdType.MESH,
            )

        for t, ref in ((0, k_ref), (1, v_ref)):
            half = ref[:, :, pl.ds(my_x * HALF, HALF), :].astype(jnp.float32)
            snd[t] = jnp.round(
                jnp.clip(half * QSCALE, -127.0, 127.0)
            ).astype(jnp.int8)

        pl.semaphore_wait(barrier_sem, 2)

        def chunk_copy(src, dst, c, send_sem, recv_sem, dev):
            t, sub = divmod(c, 2)
            sl = (t, slice(None), slice(None), pl.ds(sub * QTR, QTR))
            return pltpu.make_async_remote_copy(
                src_ref=src.at[sl], dst_ref=dst.at[sl],
                send_sem=send_sem, recv_sem=recv_sem,
                device_id=dev, device_id_type=pl.DeviceIdType.MESH,
            )

        rdma_y = [
            chunk_copy(snd, rcv_y, c, sems_ys.at[c], sems_yr.at[c], nbr_y)
            for c in range(4)
        ]
        for r in rdma_y:
            r.start()

        def block(b, h, k_blk, v_blk, s_scale):
            q = q_ref[b, h]
            s = lax.dot_general(
                q, k_blk, (((1,), (1,)), ((), ())),
                preferred_element_type=jnp.float32,
            ) * s_scale
            p = jnp.exp(s)
            l = jnp.sum(p, axis=-1, keepdims=True)
            o = lax.dot_general(
                p.astype(jnp.bfloat16), v_blk, (((1,), (0,)), ((), ())),
                preferred_element_type=jnp.float32,
            )
            return o, l

        for b in range(B):
            for h in range(H):
                o, l = block(b, h, k_ref[b, h], v_ref[b, h], SCALE)
                o_ref[b, h] = o
                l_ref[b, h] = l

        rdma_x = [
            chunk_copy(rcv_y, rcv_x, c, sems_xs.at[c], sems_xr.at[c], nbr_x)
            for c in range(4)
        ]
        for ry, rx in zip(rdma_y, rdma_x):
            ry.wait_recv()
            rx.start()

        for b in range(B):
            for h in range(H):
                o, l = block(b, h, rcv_y[0, b, h].astype(jnp.bfloat16),
                             rcv_y[1, b, h].astype(jnp.bfloat16),
                             SCALE * DEQ)
                o_ref[b, h] += o * DEQ
                l_ref[b, h] += l

        for rx in rdma_x:
            rx.wait_recv()

        for b in range(B):
            for h in range(H):
                o, l = block(b, h, rcv_x[0, b, h].astype(jnp.bfloat16),
                             rcv_x[1, b, h].astype(jnp.bfloat16),
                             SCALE * DEQ)
                o_ref[b, h] = (o_ref[b, h] + o * DEQ) \
                    * (1.0 / (l_ref[b, h] + l))

        for r in rdma_y + rdma_x:
            r.wait_send()

    out_t = pl.pallas_call(
        body,
        out_shape=jax.ShapeDtypeStruct((B, H, S, D), jnp.float32),
        in_specs=[
            pl.BlockSpec(memory_space=pltpu.VMEM),
            pl.BlockSpec(memory_space=pltpu.VMEM),
            pl.BlockSpec(memory_space=pltpu.VMEM),
        ],
        out_specs=pl.BlockSpec(memory_space=pltpu.VMEM),
        scratch_shapes=[
            pltpu.VMEM((2, B, H, HALF, D), jnp.int8),
            pltpu.VMEM((2, B, H, HALF, D), jnp.int8),
            pltpu.VMEM((2, B, H, HALF, D), jnp.int8),
            pltpu.VMEM((B, H, S, 1), jnp.float32),
            pltpu.SemaphoreType.DMA((4,)),
            pltpu.SemaphoreType.DMA((4,)),
            pltpu.SemaphoreType.DMA((4,)),
            pltpu.SemaphoreType.DMA((4,)),
        ],
        compiler_params=pltpu.CompilerParams(collective_id=0),
    )(Qt, Kt, Vt)

    return jnp.transpose(out_t, (0, 2, 1, 3))
